# baseline (speedup 1.0000x reference)
import functools

import jax
import jax.numpy as jnp
from jax import lax
from jax.experimental import pallas as pl
from jax.experimental.pallas import tpu as pltpu
from jax.experimental.pallas import tpu_sc as plsc

N = 10000
D = 128
E = 320000
NUM_LAYERS = 5
DH = D // 2

CHUNK = 64
NB = 4
NPH = 4
CHUNKS = 160
HALF = CHUNKS // NPH
EP = 32 * CHUNKS * CHUNK
DCH = 128
DCHUNKS = EP // (32 * DCH)
NP = 10240
RPT = NP // 16
R_BLK = 2000

_mesh = plsc.VectorSubcoreMesh(core_axis_name="c", subcore_axis_name="s")


@functools.partial(
    pl.kernel,
    mesh=_mesh,
    out_type=jax.ShapeDtypeStruct((2, NP, D), jnp.float32),
    scratch_types=[
        pltpu.VMEM((DCHUNKS, DCH), jnp.int32),
        pltpu.VMEM((DCH, D), jnp.float32),
        pltpu.VMEM_SHARED((NP, D), jnp.float32),
    ],
)
def _deg_sc(ones_hbm, zeros_hbm, dst_hbm, out_hbm, idx_d, ones_v, deg_sp):
    c = lax.axis_index("c")
    s = lax.axis_index("s")
    w = c * 16 + s
    r0 = s * RPT
    pltpu.sync_copy(zeros_hbm.at[pl.ds(r0, RPT)], deg_sp.at[pl.ds(r0, RPT)])
    pltpu.sync_copy(ones_hbm, ones_v)
    pltpu.sync_copy(dst_hbm.at[w], idx_d)
    plsc.subcore_barrier()

    def body(j, carry):
        pltpu.sync_copy(ones_v, deg_sp.at[idx_d.at[j]], add=True)
        return carry

    lax.fori_loop(0, DCHUNKS, body, 0)
    plsc.subcore_barrier()
    pltpu.sync_copy(deg_sp.at[pl.ds(r0, RPT)], out_hbm.at[c, pl.ds(r0, RPT)])


@functools.partial(
    pl.kernel,
    mesh=_mesh,
    out_type=jax.ShapeDtypeStruct((2, NP, D), jnp.float32),
    scratch_types=[
        pltpu.VMEM((HALF, CHUNK), jnp.int32),
        pltpu.VMEM((HALF, CHUNK), jnp.int32),
        *([pltpu.VMEM((CHUNK, D), jnp.float32)] * NB),
        pltpu.VMEM_SHARED((NP, D), jnp.float32),
        *([pltpu.SemaphoreType.DMA] * (2 * NB)),
    ],
)
def _msgpass_sc(h2_hbm, zeros_hbm, src_hbm, dst_hbm, out_hbm,
                idx_s, idx_d, *rest):
    rows = rest[:NB]
    acc_sp = rest[NB]
    gsem = rest[NB + 1:NB + 1 + NB]
    ssem = rest[NB + 1 + NB:]
    c = lax.axis_index("c")
    s = lax.axis_index("s")
    w = c * 16 + s
    r0 = s * RPT
    pltpu.sync_copy(zeros_hbm.at[pl.ds(r0, RPT)], acc_sp.at[pl.ds(r0, RPT)])
    plsc.subcore_barrier()

    table = h2_hbm.at[c]

    def _gather(j, b):
        pltpu.async_copy(table.at[idx_s.at[j]], rows[b], gsem[b])

    def _gather_wait(j, b):
        pltpu.make_async_copy(table.at[idx_s.at[j]], rows[b],
                              gsem[b]).wait()

    def _scatter(j, b):
        pltpu.async_copy(rows[b], acc_sp.at[idx_d.at[j]], ssem[b], add=True)

    def _scatter_wait(j, b):
        pltpu.make_async_copy(rows[b], acc_sp.at[idx_d.at[j]],
                              ssem[b]).wait()

    for p in range(NPH):
        pltpu.sync_copy(src_hbm.at[w, pl.ds(p * HALF, HALF)], idx_s)
        pltpu.sync_copy(dst_hbm.at[w, pl.ds(p * HALF, HALF)], idx_d)
        for b in range(NB):
            _gather(b, b)

        def outer(t, carry):
            for b in range(NB):
                j = t * NB + b
                bp = (b - 1) % NB
                _gather_wait(j, b)
                _scatter(j, b)
                @pl.when(j >= 1)
                def _():
                    _scatter_wait(j - 1, bp)

                @pl.when(jnp.logical_and(j >= 1, j + NB - 1 < HALF))
                def _():
                    _gather(j + NB - 1, bp)
            return carry

        lax.fori_loop(0, HALF // NB, outer, 0)
        _scatter_wait(HALF - 1, (HALF - 1) % NB)
    plsc.subcore_barrier()
    pltpu.sync_copy(acc_sp.at[pl.ds(r0, RPT)], out_hbm.at[c, pl.ds(r0, RPT)])


def _prelude_tc(deg2_ref, x_ref, w_ref, dinv_ref, h2_ref):
    deg = deg2_ref[0][:, 0:1] + deg2_ref[1][:, 0:1] + 1.0
    dinv = jnp.broadcast_to(lax.rsqrt(deg), (R_BLK, D))
    dinv_ref[...] = dinv
    h = jnp.dot(x_ref[...], w_ref[...], preferred_element_type=jnp.float32)
    h2 = h * dinv
    h2_ref[0] = h2
    h2_ref[1] = h2


def _layer_tc(acc2_ref, h2_ref, dinv_ref, xo_ref, w_ref, b_ref, h2o_ref):
    dinv = dinv_ref[...]
    pre = (acc2_ref[0] + acc2_ref[1] + h2_ref[0]) * dinv + b_ref[...]
    xc = jnp.maximum(pre + xo_ref[...], 0.0)
    h2n = jnp.dot(xc, w_ref[...],
                  preferred_element_type=jnp.float32) * dinv
    h2o_ref[0] = h2n
    h2o_ref[1] = h2n


def _final_tc(acc2_ref, h2_ref, dinv_ref, b_ref, out_ref):
    out_ref[...] = ((acc2_ref[0] + acc2_ref[1] + h2_ref[0])
                    * dinv_ref[...] + b_ref[...])


_GRID = (N // R_BLK,)
_spec_nd = pl.BlockSpec((R_BLK, D), lambda i: (i, 0))
_spec_acc2 = pl.BlockSpec((2, R_BLK, D), lambda i: (0, i, 0))
_spec_w = pl.BlockSpec((D, D), lambda i: (0, 0))
_spec_b = pl.BlockSpec((1, D), lambda i: (0, 0))

_spec_h2 = pl.BlockSpec((2, R_BLK, D), lambda i: (0, i, 0))
_h2_shape = jax.ShapeDtypeStruct((2, N, D), jnp.float32)

_prelude_call = pl.pallas_call(
    _prelude_tc,
    grid=_GRID,
    in_specs=[_spec_acc2, _spec_nd, _spec_w],
    out_specs=[_spec_nd, _spec_h2],
    out_shape=[jax.ShapeDtypeStruct((N, D), jnp.float32), _h2_shape],
)

_layer_call = pl.pallas_call(
    _layer_tc,
    grid=_GRID,
    in_specs=[_spec_acc2, _spec_h2, _spec_nd, _spec_nd, _spec_w, _spec_b],
    out_specs=_spec_h2,
    out_shape=_h2_shape,
)

_final_call = pl.pallas_call(
    _final_tc,
    grid=_GRID,
    in_specs=[_spec_acc2, _spec_h2, _spec_nd, _spec_b],
    out_specs=_spec_nd,
    out_shape=jax.ShapeDtypeStruct((N, D), jnp.float32),
)


def kernel(x, edge_index, W0, b0, W1, b1, W2, b2, W3, b3, W4, b4):
    pad = EP - E
    srcp = jnp.concatenate([edge_index[0], jnp.zeros((pad,), jnp.int32)])
    trash = N + (jnp.arange(pad, dtype=jnp.int32) % (NP - N))
    dstp = jnp.concatenate([edge_index[1], trash])
    src3 = srcp.reshape(32, CHUNKS, CHUNK)
    dst3 = dstp.reshape(32, CHUNKS, CHUNK)
    dst3deg = dstp.reshape(32, DCHUNKS, DCH)

    zeros_d = jnp.zeros((NP, D), jnp.float32)
    ones_d = jnp.ones((DCH, D), jnp.float32)

    deg2 = _deg_sc(ones_d, zeros_d, dst3deg)
    dinvb, h2 = _prelude_call(deg2, x, W0)

    Ws = [W1, W2, W3, W4]
    bs = [b0.reshape(1, D), b1.reshape(1, D), b2.reshape(1, D),
          b3.reshape(1, D), b4.reshape(1, D)]
    for i in range(NUM_LAYERS - 1):
        acc2 = _msgpass_sc(h2, zeros_d, src3, dst3)
        h2 = _layer_call(acc2, h2, dinvb, x, Ws[i], bs[i])
    acc2 = _msgpass_sc(h2, zeros_d, src3, dst3)
    return _final_call(acc2, h2, dinvb, bs[4])


# speedup vs baseline: 1.5309x; 1.5309x over previous
import functools

import jax
import jax.numpy as jnp
from jax import lax
from jax.experimental import pallas as pl
from jax.experimental.pallas import tpu as pltpu
from jax.experimental.pallas import tpu_sc as plsc

N = 10000
D = 128
E = 320000
NUM_LAYERS = 5
DH = D // 2

CHUNK = 128
NB = 4
CHUNKS = 160
EP = 16 * CHUNKS * CHUNK
DCH = 128
DCHUNKS = EP // (32 * DCH)
NP = 10240
RPT = NP // 16
R_BLK = 2000

_mesh = plsc.VectorSubcoreMesh(core_axis_name="c", subcore_axis_name="s")


@functools.partial(
    pl.kernel,
    mesh=_mesh,
    out_type=jax.ShapeDtypeStruct((2, NP, D), jnp.float32),
    scratch_types=[
        pltpu.VMEM((DCHUNKS, DCH), jnp.int32),
        pltpu.VMEM((DCH, D), jnp.float32),
        pltpu.VMEM_SHARED((NP, D), jnp.float32),
    ],
)
def _deg_sc(ones_hbm, zeros_hbm, dst_hbm, out_hbm, idx_d, ones_v, deg_sp):
    c = lax.axis_index("c")
    s = lax.axis_index("s")
    w = c * 16 + s
    r0 = s * RPT
    pltpu.sync_copy(zeros_hbm.at[pl.ds(r0, RPT)], deg_sp.at[pl.ds(r0, RPT)])
    pltpu.sync_copy(ones_hbm, ones_v)
    pltpu.sync_copy(dst_hbm.at[w], idx_d)
    plsc.subcore_barrier()

    def body(j, carry):
        pltpu.sync_copy(ones_v, deg_sp.at[idx_d.at[j]], add=True)
        return carry

    lax.fori_loop(0, DCHUNKS, body, 0)
    plsc.subcore_barrier()
    pltpu.sync_copy(deg_sp.at[pl.ds(r0, RPT)], out_hbm.at[c, pl.ds(r0, RPT)])


@functools.partial(
    pl.kernel,
    mesh=_mesh,
    compiler_params=pltpu.CompilerParams(use_tc_tiling_on_sc=False),
    out_type=jax.ShapeDtypeStruct((2, NP, DH), jnp.float32),
    scratch_types=[
        pltpu.VMEM((CHUNKS, CHUNK), jnp.int32),
        pltpu.VMEM((CHUNKS, CHUNK), jnp.int32),
        *([pltpu.VMEM((CHUNK, DH), jnp.float32)] * NB),
        pltpu.VMEM_SHARED((NP, DH), jnp.float32),
        *([pltpu.SemaphoreType.DMA] * (2 * NB)),
    ],
)
def _msgpass_sc(h2_hbm, zeros_hbm, src_hbm, dst_hbm, out_hbm,
                idx_s, idx_d, *rest):
    rows = rest[:NB]
    acc_sp = rest[NB]
    gsem = rest[NB + 1:NB + 1 + NB]
    ssem = rest[NB + 1 + NB:]
    c = lax.axis_index("c")
    s = lax.axis_index("s")
    r0 = s * RPT
    pltpu.sync_copy(zeros_hbm.at[pl.ds(r0, RPT)], acc_sp.at[pl.ds(r0, RPT)])
    pltpu.sync_copy(src_hbm.at[s], idx_s)
    pltpu.sync_copy(dst_hbm.at[s], idx_d)
    plsc.subcore_barrier()

    table = h2_hbm.at[c]

    def _gather(j, b):
        pltpu.async_copy(table.at[idx_s.at[j]], rows[b], gsem[b])

    def _gather_wait(j, b):
        pltpu.make_async_copy(table.at[idx_s.at[j]], rows[b], gsem[b]).wait()

    def _scatter(j, b):
        pltpu.async_copy(rows[b], acc_sp.at[idx_d.at[j]], ssem[b], add=True)

    def _scatter_wait(j, b):
        pltpu.make_async_copy(rows[b], acc_sp.at[idx_d.at[j]],
                              ssem[b]).wait()

    for b in range(NB):
        _gather(b, b)

    def outer(t, carry):
        for b in range(NB):
            j = t * NB + b
            bp = (b - 1) % NB
            _gather_wait(j, b)
            _scatter(j, b)
            @pl.when(j >= 1)
            def _():
                _scatter_wait(j - 1, bp)

            @pl.when(jnp.logical_and(j >= 1, j + NB - 1 < CHUNKS))
            def _():
                _gather(j + NB - 1, bp)
        return carry

    lax.fori_loop(0, CHUNKS // NB, outer, 0)
    _scatter_wait(CHUNKS - 1, (CHUNKS - 1) % NB)
    plsc.subcore_barrier()
    pltpu.sync_copy(acc_sp.at[pl.ds(r0, RPT)], out_hbm.at[c, pl.ds(r0, RPT)])


def _cat(ref2):
    return jnp.concatenate([ref2[0], ref2[1]], axis=1)


def _prelude_tc(deg2_ref, x_ref, w_ref, dinv_ref, h2_ref):
    deg = deg2_ref[0][:, 0:1] + deg2_ref[1][:, 0:1] + 1.0
    dinv = jnp.broadcast_to(lax.rsqrt(deg), (R_BLK, D))
    dinv_ref[...] = dinv
    h = jnp.dot(x_ref[...], w_ref[...], preferred_element_type=jnp.float32)
    h2 = h * dinv
    h2_ref[0] = h2[:, :DH]
    h2_ref[1] = h2[:, DH:]


def _layer_tc(acc2_ref, h2_ref, dinv_ref, xo_ref, w_ref, b_ref, h2o_ref):
    dinv = dinv_ref[...]
    pre = (_cat(acc2_ref) + _cat(h2_ref)) * dinv + b_ref[...]
    xc = jnp.maximum(pre + xo_ref[...], 0.0)
    h2n = jnp.dot(xc, w_ref[...], preferred_element_type=jnp.float32) * dinv
    h2o_ref[0] = h2n[:, :DH]
    h2o_ref[1] = h2n[:, DH:]


def _final_tc(acc2_ref, h2_ref, dinv_ref, b_ref, out_ref):
    out_ref[...] = ((_cat(acc2_ref) + _cat(h2_ref))
                    * dinv_ref[...] + b_ref[...])


_GRID = (N // R_BLK,)
_spec_nd = pl.BlockSpec((R_BLK, D), lambda i: (i, 0))
_spec_half2 = pl.BlockSpec((2, R_BLK, DH), lambda i: (0, i, 0))
_spec_deg2 = pl.BlockSpec((2, R_BLK, D), lambda i: (0, i, 0))
_spec_w = pl.BlockSpec((D, D), lambda i: (0, 0))
_spec_b = pl.BlockSpec((1, D), lambda i: (0, 0))

_half2_shape = jax.ShapeDtypeStruct((2, N, DH), jnp.float32)

_prelude_call = pl.pallas_call(
    _prelude_tc,
    grid=_GRID,
    in_specs=[_spec_deg2, _spec_nd, _spec_w],
    out_specs=[_spec_nd, _spec_half2],
    out_shape=[jax.ShapeDtypeStruct((N, D), jnp.float32), _half2_shape],
)

_layer_call = pl.pallas_call(
    _layer_tc,
    grid=_GRID,
    in_specs=[_spec_half2, _spec_half2, _spec_nd, _spec_nd, _spec_w, _spec_b],
    out_specs=_spec_half2,
    out_shape=_half2_shape,
)

_final_call = pl.pallas_call(
    _final_tc,
    grid=_GRID,
    in_specs=[_spec_half2, _spec_half2, _spec_nd, _spec_b],
    out_specs=_spec_nd,
    out_shape=jax.ShapeDtypeStruct((N, D), jnp.float32),
)


def kernel(x, edge_index, W0, b0, W1, b1, W2, b2, W3, b3, W4, b4):
    pad = EP - E
    srcp = jnp.concatenate([edge_index[0], jnp.zeros((pad,), jnp.int32)])
    trash = N + (jnp.arange(pad, dtype=jnp.int32) % (NP - N))
    dstp = jnp.concatenate([edge_index[1], trash])
    src3 = srcp.reshape(16, CHUNKS, CHUNK)
    dst3 = dstp.reshape(16, CHUNKS, CHUNK)
    dst3deg = dstp.reshape(32, DCHUNKS, DCH)

    zeros_d = jnp.zeros((NP, D), jnp.float32)
    zeros_h = jnp.zeros((NP, DH), jnp.float32)
    ones_d = jnp.ones((DCH, D), jnp.float32)

    deg2 = _deg_sc(ones_d, zeros_d, dst3deg)
    dinvb, h2 = _prelude_call(deg2, x, W0)

    Ws = [W1, W2, W3, W4]
    bs = [b0.reshape(1, D), b1.reshape(1, D), b2.reshape(1, D),
          b3.reshape(1, D), b4.reshape(1, D)]
    for i in range(NUM_LAYERS - 1):
        acc2 = _msgpass_sc(h2, zeros_h, src3, dst3)
        h2 = _layer_call(acc2, h2, dinvb, x, Ws[i], bs[i])
    acc2 = _msgpass_sc(h2, zeros_h, src3, dst3)
    return _final_call(acc2, h2, dinvb, bs[4])
